# Initial kernel scaffold; baseline (speedup 1.0000x reference)
import jax
import jax.numpy as jnp
from jax.experimental import pallas as pl


def kernel(gate_idx, shapes, edge_index, emb_table, W_dim, b_dim, W_l0, b_l0, W_r0, W_l1, b_l1, W_r1, W_l2, b_l2, W_r2, W_gat, att_src, att_dst, b_gat, W_e0, b_e0, W_e1, b_e1, W_out, b_out):
    raise NotImplementedError("write your pallas kernel here")


# speedup vs baseline: 5.7102x; 5.7102x over previous
import functools

import jax
import jax.numpy as jnp
from jax import lax
from jax.experimental import pallas as pl
from jax.experimental.pallas import tpu as pltpu
from jax.experimental.pallas import tpu_sc as plsc

NN = 10000
EE = 320000
HID = 128
NGATE = 16
NPAD = 10240
NC = 2
NS = 16
NW = NC * NS
EPT = EE // NW
ROWS_PT = NPAD // NS

_mesh = plsc.VectorSubcoreMesh(
    core_axis_name="c", subcore_axis_name="s", num_cores=NC, num_subcores=NS)

_f32 = jnp.float32
_sc_params = pltpu.CompilerParams(needs_layout_passes=False)


def _zero_vec_ref(ref, n):
    def body(i, _):
        ref[pl.ds(i * 16, 16)] = jnp.zeros((16,), _f32)
        return 0
    lax.fori_loop(0, n // 16, body, 0)


def _zero_rows_ref(ref, k):
    def body(i, _):
        for j in range(HID // 16):
            ref[i, pl.ds(j * 16, 16)] = jnp.zeros((16,), _f32)
        return 0
    lax.fori_loop(0, k, body, 0)


def _make_rowseg(with_deg, with_scale, chunk):
    nch = EPT // chunk
    outs = [jax.ShapeDtypeStruct((NC, NPAD, HID), _f32)]
    if with_deg:
        outs.append(jax.ShapeDtypeStruct((NW, NPAD), _f32))
    scratch = [
        pltpu.VMEM((chunk,), jnp.int32),
        pltpu.VMEM((chunk,), jnp.int32),
        pltpu.VMEM((chunk, HID), _f32),
        pltpu.VMEM_SHARED((NPAD, HID), _f32),
        pltpu.SemaphoreType.DMA,
    ]
    if with_scale:
        scratch.append(pltpu.VMEM((chunk,), _f32))
    if with_deg:
        scratch.append(pltpu.VMEM((NPAD,), _f32))

    def body(*refs):
        i = 0
        p_hbm = refs[i]; i += 1
        src_hbm = refs[i]; i += 1
        dst_hbm = refs[i]; i += 1
        scale_hbm = None
        if with_scale:
            scale_hbm = refs[i]; i += 1
        s_hbm = refs[i]; i += 1
        deg_hbm = None
        if with_deg:
            deg_hbm = refs[i]; i += 1
        sidx_v = refs[i]; i += 1
        didx_v = refs[i]; i += 1
        rows_v = refs[i]; i += 1
        acc_sh = refs[i]; i += 1
        sem = refs[i]; i += 1
        w_v = None
        if with_scale:
            w_v = refs[i]; i += 1
        deg_v = None
        if with_deg:
            deg_v = refs[i]; i += 1

        c = lax.axis_index("c")
        s = lax.axis_index("s")
        base_e = (c * NS + s) * EPT

        _zero_rows_ref(rows_v, chunk)
        def zcp(i2, _):
            pltpu.sync_copy(rows_v,
                            acc_sh.at[pl.ds(s * ROWS_PT + i2 * chunk, chunk)])
            return 0
        lax.fori_loop(0, ROWS_PT // chunk, zcp, 0)
        if with_deg:
            _zero_vec_ref(deg_v, NPAD)
        plsc.subcore_barrier()

        def step(it, _):
            off = base_e + it * chunk
            pltpu.sync_copy(src_hbm.at[pl.ds(off, chunk)], sidx_v)
            pltpu.sync_copy(dst_hbm.at[pl.ds(off, chunk)], didx_v)
            pltpu.async_copy(p_hbm.at[sidx_v], rows_v, sem).wait()
            if with_scale:
                pltpu.sync_copy(scale_hbm.at[pl.ds(off, chunk)], w_v)
                for e in range(chunk):
                    we = plsc.load_gather(
                        w_v, [jnp.full((16,), e, jnp.int32)])
                    for j in range(HID // 16):
                        rows_v[e, pl.ds(j * 16, 16)] = (
                            rows_v[e, pl.ds(j * 16, 16)] * we)
            if with_deg:
                one = jnp.ones((16,), _f32)
                for g in range(chunk // 16):
                    d16 = didx_v[pl.ds(g * 16, 16)]
                    plsc.addupdate_scatter(deg_v, [d16], one)
            pltpu.sync_copy(rows_v, acc_sh.at[didx_v], add=True)
            return 0
        lax.fori_loop(0, nch, step, 0)
        plsc.subcore_barrier()

        row0 = s * ROWS_PT
        pltpu.sync_copy(acc_sh.at[pl.ds(row0, ROWS_PT)],
                        s_hbm.at[c, pl.ds(row0, ROWS_PT)])
        if with_deg:
            pltpu.sync_copy(deg_v, deg_hbm.at[c * NS + s])

    return pl.kernel(body, out_type=tuple(outs) if len(outs) > 1 else outs[0],
                     mesh=_mesh, scratch_types=scratch,
                     compiler_params=_sc_params)


def _make_gat_scalar():
    outs = (jax.ShapeDtypeStruct((EE,), _f32),
            jax.ShapeDtypeStruct((NW, NPAD), _f32))
    scratch = [
        pltpu.VMEM((NPAD,), _f32),
        pltpu.VMEM((NPAD,), _f32),
        pltpu.VMEM((EPT,), jnp.int32),
        pltpu.VMEM((EPT,), jnp.int32),
        pltpu.VMEM((EPT,), _f32),
        pltpu.VMEM((NPAD,), _f32),
    ]

    def body(hs_hbm, hd_hbm, src_hbm, dst_hbm, ee_hbm, ssum_hbm,
             hs_v, hd_v, src_v, dst_v, ee_v, ssum_v):
        c = lax.axis_index("c")
        s = lax.axis_index("s")
        tid = c * NS + s
        base_e = tid * EPT
        pltpu.sync_copy(hs_hbm, hs_v)
        pltpu.sync_copy(hd_hbm, hd_v)
        pltpu.sync_copy(src_hbm.at[pl.ds(base_e, EPT)], src_v)
        pltpu.sync_copy(dst_hbm.at[pl.ds(base_e, EPT)], dst_v)
        _zero_vec_ref(ssum_v, NPAD)

        def step(i, _):
            s16 = src_v[pl.ds(i * 16, 16)]
            d16 = dst_v[pl.ds(i * 16, 16)]
            a = plsc.load_gather(hs_v, [s16])
            b = plsc.load_gather(hd_v, [d16])
            t = a + b
            t = jnp.where(t >= 0.0, t, t * 0.2)
            eev = jnp.exp(t)
            ee_v[pl.ds(i * 16, 16)] = eev
            plsc.addupdate_scatter(ssum_v, [d16], eev)
            return 0
        lax.fori_loop(0, EPT // 16, step, 0)

        pltpu.sync_copy(ee_v, ee_hbm.at[pl.ds(base_e, EPT)])
        pltpu.sync_copy(ssum_v, ssum_hbm.at[tid])

    return pl.kernel(body, out_type=outs, mesh=_mesh, scratch_types=scratch,
                     compiler_params=_sc_params)


def _make_edge_feat(chunk):
    nch = EPT // chunk
    out = jax.ShapeDtypeStruct((EE, HID), _f32)
    scratch = [
        pltpu.VMEM((chunk,), jnp.int32),
        pltpu.VMEM((chunk,), jnp.int32),
        pltpu.VMEM((chunk, HID), _f32),
        pltpu.VMEM((chunk, HID), _f32),
        pltpu.SemaphoreType.DMA,
        pltpu.SemaphoreType.DMA,
    ]

    def body(y_hbm, src_hbm, dst_hbm, z_hbm, sidx_v, didx_v, ra_v, rb_v,
             sema, semb):
        c = lax.axis_index("c")
        s = lax.axis_index("s")
        base_e = (c * NS + s) * EPT

        def step(it, _):
            off = base_e + it * chunk
            pltpu.sync_copy(src_hbm.at[pl.ds(off, chunk)], sidx_v)
            pltpu.sync_copy(dst_hbm.at[pl.ds(off, chunk)], didx_v)
            ca = pltpu.async_copy(y_hbm.at[sidx_v], ra_v, sema)
            cb = pltpu.async_copy(y_hbm.at[didx_v], rb_v, semb)
            ca.wait()
            cb.wait()
            for e in range(chunk):
                for j in range(HID // 16):
                    ra_v[e, pl.ds(j * 16, 16)] = (
                        ra_v[e, pl.ds(j * 16, 16)]
                        + rb_v[e, pl.ds(j * 16, 16)])
            pltpu.sync_copy(ra_v, z_hbm.at[pl.ds(off, chunk)])
            return 0
        lax.fori_loop(0, nch, step, 0)

    return pl.kernel(body, out_type=out, mesh=_mesh, scratch_types=scratch,
                     compiler_params=_sc_params)


def _dot(a, b):
    return jnp.dot(a, b, preferred_element_type=_f32)


def _tk1_body(gate_ref, shp_ref, emb_ref, wd_ref, bd_ref, wl0_ref,
              x0_ref, p0_ref):
    g = gate_ref[...]
    oh = (g == lax.broadcasted_iota(jnp.int32, (NPAD, NGATE), 1))
    xe = _dot(oh.astype(_f32), emb_ref[...])
    xd = _dot(shp_ref[...], wd_ref[...]) + bd_ref[...]
    x0 = jnp.concatenate([xd, xe], axis=1)
    x0_ref[...] = x0
    p0_ref[...] = _dot(x0, wl0_ref[...])


def _col_sum(parts):
    ones = jnp.ones((NW, 1), _f32)
    return lax.dot_general(parts, ones, (((0,), (0,)), ((), ())),
                           preferred_element_type=_f32)


def _tk2a_body(s_ref, degp_ref, x_ref, wr_ref, bl_ref, wln_ref,
               deginv_ref, x1_ref, p1_ref):
    deg = jnp.maximum(_col_sum(degp_ref[...]), 1.0)
    dinv = 1.0 / deg
    deginv_ref[...] = dinv
    agg = (s_ref[0] + s_ref[1]) * dinv
    x1 = jnp.maximum(agg + _dot(x_ref[...], wr_ref[...]) + bl_ref[...], 0.0)
    x1_ref[...] = x1
    p1_ref[...] = _dot(x1, wln_ref[...])


def _tk2b_body(s_ref, deginv_ref, x_ref, wr_ref, bl_ref, wln_ref,
               x1_ref, p1_ref):
    agg = (s_ref[0] + s_ref[1]) * deginv_ref[...]
    x1 = jnp.maximum(agg + _dot(x_ref[...], wr_ref[...]) + bl_ref[...], 0.0)
    x1_ref[...] = x1
    p1_ref[...] = _dot(x1, wln_ref[...])


def _tk2c_body(s_ref, deginv_ref, x_ref, wr_ref, bl_ref, wgat_ref,
               asrc_ref, adst_ref, h_ref, hs_ref, hd_ref):
    agg = (s_ref[0] + s_ref[1]) * deginv_ref[...]
    x3 = jnp.maximum(agg + _dot(x_ref[...], wr_ref[...]) + bl_ref[...], 0.0)
    h = _dot(x3, wgat_ref[...])
    h_ref[...] = h
    hs_ref[...] = _dot(h, asrc_ref[...])
    hd_ref[...] = _dot(h, adst_ref[...])


def _tk4_body(r_ref, ssump_ref, bgat_ref, we0_ref, be0_ref, y_ref):
    denom = jnp.maximum(_col_sum(ssump_ref[...]), 1e-16)
    x4 = jnp.maximum((r_ref[0] + r_ref[1]) / denom + bgat_ref[...], 0.0)
    y_ref[...] = _dot(x4, we0_ref[...]) + 0.5 * be0_ref[...]


_BE = 2560


def _tk5_body(z_ref, we1_ref, be1_ref, wo_ref, bo_ref, o_ref):
    xe1 = jnp.maximum(z_ref[...], 0.0)
    xe2 = jnp.maximum(_dot(xe1, we1_ref[...]) + be1_ref[...], 0.0)
    o_ref[...] = _dot(xe2, wo_ref[...]) + bo_ref[...]


def _tc(body, out_shape):
    return pl.pallas_call(body, out_shape=out_shape)


def kernel(gate_idx, shapes, edge_index, emb_table, W_dim, b_dim,
           W_l0, b_l0, W_r0, W_l1, b_l1, W_r1, W_l2, b_l2, W_r2,
           W_gat, att_src, att_dst, b_gat,
           W_e0, b_e0, W_e1, b_e1, W_out, b_out):
    padn = NPAD - NN
    gate_p = jnp.pad(gate_idx.astype(jnp.int32), (0, padn)).reshape(NPAD, 1)
    shp_p = jnp.pad(shapes, ((0, padn), (0, 0)))
    ei = edge_index.astype(jnp.int32)
    src = ei[0]
    dst = ei[1]
    nd = jax.ShapeDtypeStruct((NPAD, HID), _f32)
    nd1 = jax.ShapeDtypeStruct((NPAD, 1), _f32)

    r2 = lambda v: v.reshape(1, -1)

    x0, p0 = _tc(_tk1_body, (nd, nd))(
        gate_p, shp_p, emb_table, W_dim, r2(b_dim), W_l0)

    seg_deg = _make_rowseg(with_deg=True, with_scale=False, chunk=80)
    seg = _make_rowseg(with_deg=False, with_scale=False, chunk=80)
    seg_scaled = _make_rowseg(with_deg=False, with_scale=True, chunk=40)

    s0, degp = seg_deg(p0, src, dst)
    deginv, x1, p1 = _tc(_tk2a_body, (nd1, nd, nd))(
        s0, degp, x0, W_r0, r2(b_l0), W_l1)

    s1 = seg(p1, src, dst)
    x2, p2 = _tc(_tk2b_body, (nd, nd))(
        s1, deginv, x1, W_r1, r2(b_l1), W_l2)

    s2 = seg(p2, src, dst)
    h, hs, hd = _tc(_tk2c_body, (nd, nd1, nd1))(
        s2, deginv, x2, W_r2, r2(b_l2), W_gat,
        att_src.reshape(HID, 1), att_dst.reshape(HID, 1))

    ee, ssump = _make_gat_scalar()(
        hs.reshape(NPAD), hd.reshape(NPAD), src, dst)
    r = seg_scaled(h, src, dst, ee)

    y = _tc(_tk4_body, nd)(r, ssump, r2(b_gat), W_e0, r2(b_e0))
    z = _make_edge_feat(chunk=40)(y, src, dst)

    grid = (EE // _BE,)
    scores = pl.pallas_call(
        _tk5_body,
        out_shape=jax.ShapeDtypeStruct((EE, 1), _f32),
        grid=grid,
        in_specs=[
            pl.BlockSpec((_BE, HID), lambda i: (i, 0)),
            pl.BlockSpec((HID, HID), lambda i: (0, 0)),
            pl.BlockSpec((1, HID), lambda i: (0, 0)),
            pl.BlockSpec((HID, 1), lambda i: (0, 0)),
            pl.BlockSpec((1, 1), lambda i: (0, 0)),
        ],
        out_specs=pl.BlockSpec((_BE, 1), lambda i: (i, 0)),
    )(z, W_e1, r2(b_e1), W_out, b_out.reshape(1, 1))

    return scores
